# baseline (speedup 1.0000x reference)
import functools

import jax
import jax.numpy as jnp
from jax.experimental import pallas as pl
from jax.experimental.pallas import tpu as pltpu

B, C_IN, C_OUT, K = 8, 16, 32, 3
D_IN, H_IN, W_IN = 34, 66, 66
H_OUT, W_OUT = 64, 64
N_SPATIAL = H_OUT * W_OUT
N_J = 16


def _im2col(xs):
    parts = []
    for kh in range(K):
        for kw in range(K):
            parts.append(xs[kh:kh + H_OUT, kw:kw + W_OUT, :])
    p = jnp.concatenate(parts, axis=-1)
    return p.reshape(N_SPATIAL, K * K * C_IN)


def _kernel(xa_ref, xb_ref, w_ref, cb_ref, out_ref):
    j = pl.program_id(1)

    def slab(s):
        ref = xa_ref if s < 2 else xb_ref
        return ref[0, s % 2]

    wexp = w_ref[...]

    y0 = jnp.zeros((N_SPATIAL, C_OUT), jnp.float32)
    y1 = jnp.zeros((N_SPATIAL, C_OUT), jnp.float32)
    for s in range(4):
        patches = _im2col(slab(s))
        z = jnp.dot(patches, wexp, preferred_element_type=jnp.float32)
        if s <= 2:
            y0 = y0 + z[:, 32 * s:32 * s + 32]
        if s >= 1:
            y1 = y1 + z[:, 32 * (s - 1):32 * (s - 1) + 32]

    cb = cb_ref[...]
    m = jnp.maximum(y0, y1) + cb
    m4 = m.reshape(H_OUT // 2, 2, W_OUT // 2, 2, C_OUT)
    pooled = jnp.max(jnp.max(m4, axis=3), axis=1)
    csum = jnp.sum(pooled, axis=(0, 1)).reshape(1, C_OUT)

    @pl.when(j == 0)
    def _():
        out_ref[...] = jnp.zeros((1, 1, C_OUT), jnp.float32)

    out_ref[...] += csum.reshape(1, 1, C_OUT)


@jax.jit
def kernel(x, conv_weight, conv_bias, bias):
    xt = x.transpose(0, 2, 3, 4, 1).astype(jnp.bfloat16)
    wexp = conv_weight.transpose(3, 4, 1, 2, 0).reshape(
        K * K * C_IN, K * C_OUT).astype(jnp.bfloat16)
    cb = conv_bias.reshape(1, C_OUT)

    acc = pl.pallas_call(
        _kernel,
        grid=(B, N_J),
        in_specs=[
            pl.BlockSpec((1, 2, H_IN, W_IN, C_IN), lambda b, j: (b, j, 0, 0, 0)),
            pl.BlockSpec((1, 2, H_IN, W_IN, C_IN), lambda b, j: (b, j + 1, 0, 0, 0)),
            pl.BlockSpec((K * K * C_IN, K * C_OUT), lambda b, j: (0, 0)),
            pl.BlockSpec((1, C_OUT), lambda b, j: (0, 0)),
        ],
        out_specs=pl.BlockSpec((1, 1, C_OUT), lambda b, j: (b, 0, 0)),
        out_shape=jax.ShapeDtypeStruct((B, 1, C_OUT), jnp.float32),
        compiler_params=pltpu.CompilerParams(
            dimension_semantics=("parallel", "arbitrary"),
        ),
    )(xt, xt, wexp, cb)

    return (acc.sum(axis=(1, 2)) * 0.5 + bias.sum()).reshape(B, 1, 1, 1)


# speedup vs baseline: 4.1691x; 4.1691x over previous
import jax
import jax.numpy as jnp
from jax.experimental import pallas as pl
from jax.experimental.pallas import tpu as pltpu

B, C_IN, C_OUT, K = 8, 16, 32, 3
D_IN, H_IN, W_IN = 34, 66, 66
H_OUT, W_OUT = 64, 64
HW = H_IN * W_IN
HW_PAD = 4480
N_LANES = H_OUT * W_IN
N_J = 16
OFFS = tuple(kh * W_IN + kw for kh in range(K) for kw in range(K))


def _kernel(x0_ref, x1_ref, x2_ref, x3_ref, w_ref, cb_ref, mask_ref,
            out_ref, yp_ref):
    j = pl.program_id(1)
    wm = w_ref[...]

    def im2col(ref):
        xs = ref[0, 0]
        return jnp.concatenate(
            [xs[:, off:off + N_LANES] for off in OFFS], axis=0)

    def conv_pair(ra, rb):
        a = jnp.concatenate([im2col(ra), im2col(rb)], axis=1)
        y = jnp.dot(wm, a, preferred_element_type=jnp.float32)
        return y[:, :N_LANES], y[:, N_LANES:]

    @pl.when(j == 0)
    def _():
        ya, yb = conv_pair(x0_ref, x1_ref)
        yp_ref[0] = ya
        yp_ref[1] = yb

    y2, y3 = conv_pair(x2_ref, x3_ref)
    y0 = yp_ref[0, 0:32] + yp_ref[1, 32:64] + y2[64:96]
    y1 = yp_ref[1, 0:32] + y2[32:64] + y3[64:96]
    yp_ref[0] = y2
    yp_ref[1] = y3

    m = jnp.maximum(y0, y1) + cb_ref[...]
    ms1 = jnp.concatenate([m[:, 1:], m[:, :1]], axis=1)
    ma = jnp.maximum(m, ms1)
    ms66 = jnp.concatenate([ma[:, W_IN:], ma[:, :W_IN]], axis=1)
    mb = jnp.maximum(ma, ms66)
    csum = jnp.dot(mb.astype(jnp.bfloat16), mask_ref[...],
                   preferred_element_type=jnp.float32)

    @pl.when(j == 0)
    def _():
        out_ref[...] = jnp.zeros((1, C_OUT, 128), jnp.float32)

    out_ref[...] += csum.reshape(1, C_OUT, 128)


@jax.jit
def kernel(x, conv_weight, conv_bias, bias):
    x5 = jnp.pad(
        x.reshape(B, C_IN, D_IN, HW).transpose(0, 2, 1, 3).astype(jnp.bfloat16),
        ((0, 0), (0, 0), (0, 0), (0, HW_PAD - HW)))
    wm = conv_weight.transpose(2, 0, 3, 4, 1).reshape(
        K * C_OUT, K * K * C_IN).astype(jnp.bfloat16)
    cb = conv_bias.reshape(C_OUT, 1)

    lane = jnp.arange(N_LANES, dtype=jnp.int32)
    h, w = lane // W_IN, lane % W_IN
    keep = (h % 2 == 0) & (w % 2 == 0) & (w < W_OUT)
    maskc = jnp.where(keep[:, None], jnp.ones((1,), jnp.bfloat16),
                      jnp.zeros((1,), jnp.bfloat16))
    maskc = jnp.broadcast_to(maskc, (N_LANES, 128))

    slab_spec = [
        pl.BlockSpec((1, 1, C_IN, HW_PAD), lambda b, j: (b, 0, 0, 0)),
        pl.BlockSpec((1, 1, C_IN, HW_PAD), lambda b, j: (b, 1, 0, 0)),
        pl.BlockSpec((1, 1, C_IN, HW_PAD), lambda b, j: (b, 2 * j + 2, 0, 0)),
        pl.BlockSpec((1, 1, C_IN, HW_PAD), lambda b, j: (b, 2 * j + 3, 0, 0)),
    ]
    acc = pl.pallas_call(
        _kernel,
        grid=(B, N_J),
        in_specs=slab_spec + [
            pl.BlockSpec((K * C_OUT, K * K * C_IN), lambda b, j: (0, 0)),
            pl.BlockSpec((C_OUT, 1), lambda b, j: (0, 0)),
            pl.BlockSpec((N_LANES, 128), lambda b, j: (0, 0)),
        ],
        out_specs=pl.BlockSpec((1, C_OUT, 128), lambda b, j: (b, 0, 0)),
        out_shape=jax.ShapeDtypeStruct((B, C_OUT, 128), jnp.float32),
        scratch_shapes=[pltpu.VMEM((2, 3 * C_OUT, N_LANES), jnp.float32)],
        compiler_params=pltpu.CompilerParams(
            dimension_semantics=("parallel", "arbitrary"),
        ),
    )(x5, x5, x5, x5, wm, cb, maskc)

    return (acc[:, :, 0].sum(axis=1) * 0.5 + bias.sum()).reshape(B, 1, 1, 1)
